# baseline (speedup 1.0000x reference)
import functools

import jax
import jax.numpy as jnp
from jax import lax
from jax.experimental import pallas as pl
from jax.experimental.pallas import tpu as pltpu
from jax.experimental.pallas import tpu_sc as plsc

N_ROWS = 100000
D = 256
M = 15000

NC = 2
NS = 16
NW = NC * NS
LANES = 16
NJ = D // LANES

M_PAD = 15360
CHUNK = 120
NCH0 = 6
NCH1 = 2
PER_S = (NCH0 + NCH1) * CHUNK

ROWS_BLK = 3000


def _pred_row_body(t_ref, w_ref, b_ref, o_ref):
    o_ref[...] = (
        jnp.dot(t_ref[...], w_ref[...], preferred_element_type=jnp.float32)
        + b_ref[...]
    )


def _bcast_body(p_ref, o_ref):
    o_ref[...] = jnp.broadcast_to(p_ref[...], o_ref.shape)


_sc_mesh = plsc.VectorSubcoreMesh(core_axis_name="c", subcore_axis_name="s")


UNROLL = 4


OUT_W = D + LANES


@functools.partial(
    pl.kernel,
    mesh=_sc_mesh,
    out_type=jax.ShapeDtypeStruct((NW, OUT_W), jnp.float32),
    scratch_types=[
        pltpu.VMEM((NCH0, CHUNK), jnp.int32),
        pltpu.VMEM((2, CHUNK, D), jnp.float32),
        pltpu.VMEM((OUT_W,), jnp.float32),
        pltpu.SemaphoreType.DMA,
        pltpu.SemaphoreType.DMA,
    ],
)
def _sc_gather_moments(x_hbm, idx_hbm, out_hbm, idx_v, rows_v, part_v, sem0, sem1):
    cid = lax.axis_index("c")
    sid = lax.axis_index("s")
    wid = sid * NC + cid
    sems = (sem0, sem1)

    def work(nch, ch_lo):
        pltpu.sync_copy(idx_hbm.at[sid, pl.ds(ch_lo, nch)], idx_v.at[pl.ds(0, nch)])

        cps = [None] * nch
        for c in range(min(nch, 2)):
            cps[c] = pltpu.async_copy(
                x_hbm.at[idx_v.at[c]], rows_v.at[c % 2], sems[c % 2]
            )

        zeros = jnp.zeros((LANES,), jnp.float32)
        accs = tuple(zeros for _ in range(2 * NJ))
        for c in range(nch):
            cps[c].wait()
            buf = c % 2

            def body(i, accs, buf=buf):
                s = list(accs[:NJ])
                q = list(accs[NJ:])
                r0 = i * UNROLL
                for u in range(UNROLL):
                    for j in range(NJ):
                        v = rows_v[buf, r0 + u, pl.ds(j * LANES, LANES)]
                        s[j] = s[j] + v
                        q[j] = q[j] + v * v
                return tuple(s) + tuple(q)

            accs = lax.fori_loop(0, CHUNK // UNROLL, body, accs)
            if c + 2 < nch:
                cps[c + 2] = pltpu.async_copy(
                    x_hbm.at[idx_v.at[c + 2]], rows_v.at[buf], sems[buf]
                )

        base = sid * PER_S + ch_lo * CHUNK
        n_pad = jnp.clip(base + nch * CHUNK - M, 0, nch * CHUNK).astype(jnp.float32)
        qtot = zeros
        for j in range(NJ):
            v = rows_v[(nch - 1) % 2, CHUNK - 1, pl.ds(j * LANES, LANES)]
            part_v[pl.ds(j * LANES, LANES)] = accs[j] - n_pad * v
            qtot = qtot + (accs[NJ + j] - n_pad * v * v)
        part_v[pl.ds(D, LANES)] = qtot
        pltpu.sync_copy(part_v, out_hbm.at[wid])

    @pl.when(cid == 0)
    def _():
        work(NCH0, 0)

    @pl.when(cid == 1)
    def _():
        work(NCH1, NCH0)


def kernel(x, mask_idx, mask_token, W_cont, b_cont):
    p_row = pl.pallas_call(
        _pred_row_body,
        out_shape=jax.ShapeDtypeStruct((1, D), jnp.float32),
    )(mask_token.reshape(1, D), W_cont, b_cont.reshape(1, D))

    pred_cont = pl.pallas_call(
        _bcast_body,
        grid=(M // ROWS_BLK,),
        in_specs=[pl.BlockSpec((1, D), lambda i: (0, 0))],
        out_specs=pl.BlockSpec((ROWS_BLK, D), lambda i: (i, 0)),
        out_shape=jax.ShapeDtypeStruct((M, D), jnp.float32),
    )(p_row)

    idx_pad = jnp.concatenate(
        [mask_idx, jnp.zeros((M_PAD - M,), jnp.int32)]
    ).reshape(NS, NCH0 + NCH1, CHUNK)
    partials = _sc_gather_moments(x, idx_pad)

    p_vec = p_row.reshape(D)
    s_vec = jnp.sum(partials[:, :D], axis=0)
    q_tot = jnp.sum(partials[:, D:])
    total_loss = (
        q_tot - 2.0 * jnp.dot(s_vec, p_vec) + M * jnp.dot(p_vec, p_vec)
    ) / (M * D)
    return (total_loss, pred_cont)


# speedup vs baseline: 1.0990x; 1.0990x over previous
import functools

import jax
import jax.numpy as jnp
from jax import lax
from jax.experimental import pallas as pl
from jax.experimental.pallas import tpu as pltpu
from jax.experimental.pallas import tpu_sc as plsc

N_ROWS = 100000
D = 256
M = 15000

NC = 2
NS = 16
NW = NC * NS
LANES = 16
NJ = D // LANES

M_PAD = 15360
CHUNK = 120
NCH0 = 6
NCH1 = 2
PER_S = (NCH0 + NCH1) * CHUNK

ROWS_BLK = 3000


def _pred_row_body(t_ref, w_ref, b_ref, o_ref):
    o_ref[...] = (
        jnp.dot(t_ref[...], w_ref[...], preferred_element_type=jnp.float32)
        + b_ref[...]
    )


def _bcast_body(p_ref, o_ref):
    o_ref[...] = jnp.broadcast_to(p_ref[...], o_ref.shape)


def _loss_body(part_ref, p_ref, o_ref):
    pp = part_ref[...]
    p = p_ref[...]
    s = jnp.sum(pp[:, :D], axis=0, keepdims=True)
    q = jnp.sum(pp[:, D:])
    loss = (q - 2.0 * jnp.sum(s * p) + M * jnp.sum(p * p)) * (1.0 / (M * D))
    o_ref[...] = loss.reshape(1, 1)


_sc_mesh = plsc.VectorSubcoreMesh(core_axis_name="c", subcore_axis_name="s")


UNROLL = 4


OUT_W = D + LANES


@functools.partial(
    pl.kernel,
    mesh=_sc_mesh,
    out_type=jax.ShapeDtypeStruct((NW, OUT_W), jnp.float32),
    scratch_types=[
        pltpu.VMEM((NCH0, CHUNK), jnp.int32),
        pltpu.VMEM((2, CHUNK, D), jnp.float32),
        pltpu.VMEM((OUT_W,), jnp.float32),
        pltpu.SemaphoreType.DMA,
        pltpu.SemaphoreType.DMA,
    ],
)
def _sc_gather_moments(x_hbm, idx_hbm, out_hbm, idx_v, rows_v, part_v, sem0, sem1):
    cid = lax.axis_index("c")
    sid = lax.axis_index("s")
    wid = sid * NC + cid
    sems = (sem0, sem1)

    def work(nch, ch_lo):
        pltpu.sync_copy(idx_hbm.at[sid, pl.ds(ch_lo, nch)], idx_v.at[pl.ds(0, nch)])

        cps = [None] * nch
        for c in range(min(nch, 2)):
            cps[c] = pltpu.async_copy(
                x_hbm.at[idx_v.at[c]], rows_v.at[c % 2], sems[c % 2]
            )

        zeros = jnp.zeros((LANES,), jnp.float32)
        accs = tuple(zeros for _ in range(2 * NJ))
        for c in range(nch):
            cps[c].wait()
            buf = c % 2

            def body(i, accs, buf=buf):
                s = list(accs[:NJ])
                q = list(accs[NJ:])
                r0 = i * UNROLL
                for u in range(UNROLL):
                    for j in range(NJ):
                        v = rows_v[buf, r0 + u, pl.ds(j * LANES, LANES)]
                        s[j] = s[j] + v
                        q[j] = q[j] + v * v
                return tuple(s) + tuple(q)

            accs = lax.fori_loop(0, CHUNK // UNROLL, body, accs)
            if c + 2 < nch:
                cps[c + 2] = pltpu.async_copy(
                    x_hbm.at[idx_v.at[c + 2]], rows_v.at[buf], sems[buf]
                )

        base = sid * PER_S + ch_lo * CHUNK
        n_pad = jnp.clip(base + nch * CHUNK - M, 0, nch * CHUNK).astype(jnp.float32)
        qtot = zeros
        for j in range(NJ):
            v = rows_v[(nch - 1) % 2, CHUNK - 1, pl.ds(j * LANES, LANES)]
            part_v[pl.ds(j * LANES, LANES)] = accs[j] - n_pad * v
            qtot = qtot + (accs[NJ + j] - n_pad * v * v)
        part_v[pl.ds(D, LANES)] = qtot
        pltpu.sync_copy(part_v, out_hbm.at[wid])

    @pl.when(cid == 0)
    def _():
        work(NCH0, 0)

    @pl.when(cid == 1)
    def _():
        work(NCH1, NCH0)


def kernel(x, mask_idx, mask_token, W_cont, b_cont):
    p_row = pl.pallas_call(
        _pred_row_body,
        out_shape=jax.ShapeDtypeStruct((1, D), jnp.float32),
    )(mask_token.reshape(1, D), W_cont, b_cont.reshape(1, D))

    pred_cont = pl.pallas_call(
        _bcast_body,
        grid=(M // ROWS_BLK,),
        in_specs=[pl.BlockSpec((1, D), lambda i: (0, 0))],
        out_specs=pl.BlockSpec((ROWS_BLK, D), lambda i: (i, 0)),
        out_shape=jax.ShapeDtypeStruct((M, D), jnp.float32),
    )(p_row)

    idx_pad = jnp.concatenate(
        [mask_idx, jnp.zeros((M_PAD - M,), jnp.int32)]
    ).reshape(NS, NCH0 + NCH1, CHUNK)
    partials = _sc_gather_moments(x, idx_pad)

    loss = pl.pallas_call(
        _loss_body,
        out_shape=jax.ShapeDtypeStruct((1, 1), jnp.float32),
    )(partials, p_row)

    return (loss.reshape(()), pred_cont)


# speedup vs baseline: 1.1197x; 1.0188x over previous
import functools

import jax
import jax.numpy as jnp
from jax import lax
from jax.experimental import pallas as pl
from jax.experimental.pallas import tpu as pltpu
from jax.experimental.pallas import tpu_sc as plsc

N_ROWS = 100000
D = 256
M = 15000

NC = 2
NS = 16
NW = NC * NS
LANES = 16
NJ = D // LANES

CHUNK = 120
NCH0 = 6
NCH1 = 2
PER_S = (NCH0 + NCH1) * CHUNK

ROWS_BLK = 3000


def _pred_row_body(t_ref, w_ref, b_ref, o_ref):
    o_ref[...] = (
        jnp.dot(t_ref[...], w_ref[...], preferred_element_type=jnp.float32)
        + b_ref[...]
    )


def _bcast_body(p_ref, o_ref):
    o_ref[...] = jnp.broadcast_to(p_ref[...], o_ref.shape)


def _loss_body(part_ref, p_ref, o_ref):
    pp = part_ref[...]
    p = p_ref[...]
    s = jnp.sum(pp[:, :D], axis=0, keepdims=True)
    q = jnp.sum(pp[:, D:])
    loss = (q - 2.0 * jnp.sum(s * p) + M * jnp.sum(p * p)) * (1.0 / (M * D))
    o_ref[...] = loss.reshape(1, 1)


_sc_mesh = plsc.VectorSubcoreMesh(core_axis_name="c", subcore_axis_name="s")


UNROLL = 4


OUT_W = D + LANES


@functools.partial(
    pl.kernel,
    mesh=_sc_mesh,
    out_type=jax.ShapeDtypeStruct((NW, OUT_W), jnp.float32),
    scratch_types=[
        pltpu.VMEM((NCH0 * CHUNK,), jnp.int32),
        pltpu.VMEM((2, CHUNK, D), jnp.float32),
        pltpu.VMEM((OUT_W,), jnp.float32),
        pltpu.SemaphoreType.DMA,
        pltpu.SemaphoreType.DMA,
    ],
)
def _sc_gather_moments(x_hbm, idx_hbm, zpad_hbm, out_hbm, idx_v, rows_v, part_v, sem0, sem1):
    cid = lax.axis_index("c")
    sid = lax.axis_index("s")
    wid = sid * NC + cid
    sems = (sem0, sem1)

    def work(nch, ch_lo):
        n = nch * CHUNK
        last_base = (NS - 1) * PER_S + ch_lo * CHUNK
        n_real_last = max(0, min(n, M - last_base))

        @pl.when(sid < NS - 1)
        def _():
            base = pl.multiple_of(sid * PER_S + ch_lo * CHUNK, 8)
            pltpu.sync_copy(idx_hbm.at[pl.ds(base, n)], idx_v.at[pl.ds(0, n)])

        @pl.when(sid == NS - 1)
        def _():
            if n_real_last > 0:
                pltpu.sync_copy(
                    idx_hbm.at[pl.ds(last_base, n_real_last)],
                    idx_v.at[pl.ds(0, n_real_last)],
                )
            if n_real_last < n:
                pltpu.sync_copy(
                    zpad_hbm.at[pl.ds(0, n - n_real_last)],
                    idx_v.at[pl.ds(n_real_last, n - n_real_last)],
                )

        cps = [None] * nch
        for c in range(min(nch, 2)):
            cps[c] = pltpu.async_copy(
                x_hbm.at[idx_v.at[pl.ds(c * CHUNK, CHUNK)]],
                rows_v.at[c % 2],
                sems[c % 2],
            )

        zeros = jnp.zeros((LANES,), jnp.float32)
        accs = tuple(zeros for _ in range(2 * NJ))
        for c in range(nch):
            cps[c].wait()
            buf = c % 2

            def body(i, accs, buf=buf):
                s = list(accs[:NJ])
                q = list(accs[NJ:])
                r0 = i * UNROLL
                for u in range(UNROLL):
                    for j in range(NJ):
                        v = rows_v[buf, r0 + u, pl.ds(j * LANES, LANES)]
                        s[j] = s[j] + v
                        q[j] = q[j] + v * v
                return tuple(s) + tuple(q)

            accs = lax.fori_loop(0, CHUNK // UNROLL, body, accs)
            if c + 2 < nch:
                cps[c + 2] = pltpu.async_copy(
                    x_hbm.at[idx_v.at[pl.ds((c + 2) * CHUNK, CHUNK)]],
                    rows_v.at[buf],
                    sems[buf],
                )

        base = sid * PER_S + ch_lo * CHUNK
        n_pad = jnp.clip(base + nch * CHUNK - M, 0, nch * CHUNK).astype(jnp.float32)
        qtot = zeros
        for j in range(NJ):
            v = rows_v[(nch - 1) % 2, CHUNK - 1, pl.ds(j * LANES, LANES)]
            part_v[pl.ds(j * LANES, LANES)] = accs[j] - n_pad * v
            qtot = qtot + (accs[NJ + j] - n_pad * v * v)
        part_v[pl.ds(D, LANES)] = qtot
        pltpu.sync_copy(part_v, out_hbm.at[wid])

    @pl.when(cid == 0)
    def _():
        work(NCH0, 0)

    @pl.when(cid == 1)
    def _():
        work(NCH1, NCH0)


def kernel(x, mask_idx, mask_token, W_cont, b_cont):
    p_row = pl.pallas_call(
        _pred_row_body,
        out_shape=jax.ShapeDtypeStruct((1, D), jnp.float32),
    )(mask_token.reshape(1, D), W_cont, b_cont.reshape(1, D))

    pred_cont = pl.pallas_call(
        _bcast_body,
        grid=(M // ROWS_BLK,),
        in_specs=[pl.BlockSpec((1, D), lambda i: (0, 0))],
        out_specs=pl.BlockSpec((ROWS_BLK, D), lambda i: (i, 0)),
        out_shape=jax.ShapeDtypeStruct((M, D), jnp.float32),
    )(p_row)

    zpad = jnp.zeros((NCH1 * CHUNK,), jnp.int32)
    partials = _sc_gather_moments(x, mask_idx, zpad)

    loss = pl.pallas_call(
        _loss_body,
        out_shape=jax.ShapeDtypeStruct((1, 1), jnp.float32),
    )(partials, p_row)

    return (loss.reshape(()), pred_cont)
